# Initial kernel scaffold; baseline (speedup 1.0000x reference)
import jax
import jax.numpy as jnp
from jax.experimental import pallas as pl


def kernel(x, V, r, Y, edge_src, W1, W2a, W2b, W2c, Wv):
    raise NotImplementedError("write your pallas kernel here")


# speedup vs baseline: 15.7321x; 15.7321x over previous
import functools
import math

import jax
import jax.numpy as jnp
from jax import lax
from jax.experimental import pallas as pl
from jax.experimental.pallas import tpu as pltpu
from jax.experimental.pallas import tpu_sc as plsc

N_NODES = 10000
N_EDGES = 320000
FEAT = 128
N_MUL = 8
NUM_NEIGHBORS = 32.0

BLK = 2000

SC_CORES = 2
SC_SUBCORES = 16
EDGES_PER_TILE = N_EDGES // SC_SUBCORES
CHUNK = 80
BIG = 800
K_PER_BIG = BIG // CHUNK
N_BIG = EDGES_PER_TILE // BIG
N_CHUNKS = EDGES_PER_TILE // CHUNK


def _phase_a_body(x_ref, y_ref, w1t_ref, o0_ref, o1_ref):
    x = x_ref[...]
    w = jnp.dot(x, w1t_ref[...], preferred_element_type=jnp.float32)
    y = y_ref[...]
    yrep = jnp.concatenate(
        [jnp.broadcast_to(y[:, k:k + 1], (x.shape[0], 8)) for k in range(4)],
        axis=1)
    wy = w * yrep
    o0_ref[...] = wy[:, 0:16]
    o1_ref[...] = wy[:, 16:32]


def _phase_a(x, y, w1t):
    grid = (N_EDGES // BLK,)
    return pl.pallas_call(
        _phase_a_body,
        grid=grid,
        in_specs=[
            pl.BlockSpec((BLK, FEAT), lambda i: (i, 0)),
            pl.BlockSpec((BLK, 4), lambda i: (i, 0)),
            pl.BlockSpec((FEAT, 4 * N_MUL), lambda i: (0, 0)),
        ],
        out_specs=[
            pl.BlockSpec((BLK, 16), lambda i: (i, 0)),
            pl.BlockSpec((BLK, 16), lambda i: (i, 0)),
        ],
        out_shape=[
            jax.ShapeDtypeStruct((N_EDGES, 16), jnp.float32),
            jax.ShapeDtypeStruct((N_EDGES, 16), jnp.float32),
        ],
    )(x, y, w1t)


def _sc_tile_work(cid_side, wy_hbm, src2d_hbm, zeros_hbm, ns_hbm, g_hbm,
                  idx_v, buf_v, acc_sh):
    sid = lax.axis_index("s")
    rows_per_tile = N_NODES // SC_SUBCORES

    pltpu.sync_copy(src2d_hbm.at[pl.ds(sid * N_CHUNKS, N_CHUNKS), :], idx_v)

    pltpu.sync_copy(
        zeros_hbm.at[pl.ds(sid * rows_per_tile, rows_per_tile), :],
        acc_sh.at[pl.ds(sid * rows_per_tile, rows_per_tile), :])
    plsc.subcore_barrier()

    @pl.loop(0, N_BIG)
    def _(bi):
        base = sid * EDGES_PER_TILE + bi * BIG
        pltpu.sync_copy(wy_hbm.at[pl.ds(base, BIG), :], buf_v)
        for k in range(K_PER_BIG):
            pltpu.sync_copy(
                buf_v.at[pl.ds(k * CHUNK, CHUNK), :],
                acc_sh.at[idx_v.at[bi * K_PER_BIG + k]],
                add=True)

    plsc.subcore_barrier()

    pltpu.sync_copy(
        acc_sh.at[pl.ds(sid * rows_per_tile, rows_per_tile), :],
        ns_hbm.at[pl.ds(sid * rows_per_tile, rows_per_tile), :])
    plsc.subcore_barrier()

    @pl.loop(0, N_BIG)
    def _(bi):
        base = sid * EDGES_PER_TILE + bi * BIG
        for k in range(K_PER_BIG):
            pltpu.sync_copy(
                ns_hbm.at[idx_v.at[bi * K_PER_BIG + k]],
                buf_v.at[pl.ds(k * CHUNK, CHUNK), :])
        pltpu.sync_copy(buf_v, g_hbm.at[pl.ds(base, BIG), :])

    plsc.subcore_barrier()
    pltpu.sync_copy(acc_sh.at[pl.ds(sid * rows_per_tile, 8), :],
                    buf_v.at[pl.ds(0, 8), :])
    plsc.subcore_barrier()


def _sc_segsum(wy0, wy1, src2d, zeros):
    mesh = plsc.VectorSubcoreMesh(core_axis_name="c", subcore_axis_name="s")

    @functools.partial(
        pl.kernel,
        mesh=mesh,
        compiler_params=pltpu.CompilerParams(use_tc_tiling_on_sc=False,
                                             has_side_effects=True),
        out_type=[
            jax.ShapeDtypeStruct((N_NODES, 16), jnp.float32),
            jax.ShapeDtypeStruct((N_NODES, 16), jnp.float32),
            jax.ShapeDtypeStruct((N_EDGES, 16), jnp.float32),
            jax.ShapeDtypeStruct((N_EDGES, 16), jnp.float32),
        ],
        scratch_types=[
            pltpu.VMEM((N_CHUNKS, CHUNK), jnp.int32),
            pltpu.VMEM((BIG, 16), jnp.float32),
            pltpu.VMEM_SHARED((N_NODES, 16), jnp.float32),
        ],
    )
    def k(wy0_hbm, wy1_hbm, src2d_hbm, zeros_hbm,
          ns0_hbm, ns1_hbm, g0_hbm, g1_hbm, idx_v, buf_v, acc_sh):
        cid = lax.axis_index("c")

        @pl.when(cid == 0)
        def _():
            _sc_tile_work(0, wy0_hbm, src2d_hbm, zeros_hbm, ns0_hbm, g0_hbm,
                          idx_v, buf_v, acc_sh)

        @pl.when(cid == 1)
        def _():
            _sc_tile_work(1, wy1_hbm, src2d_hbm, zeros_hbm, ns1_hbm, g1_hbm,
                          idx_v, buf_v, acc_sh)

    return k(wy0, wy1, src2d, zeros)


def _phase_b_body(x_ref, g0_ref, g1_ref, v0_ref, v1_ref, r_ref,
                  w2ax_ref, w2as_ref, w2b_ref, w2c_ref, wv2_ref,
                  xo_ref, vox_ref, voy_ref, voz_ref):
    x = x_ref[...]
    g0 = g0_ref[...]
    g1 = g1_ref[...]
    v0 = v0_ref[...]
    v1 = v1_ref[...]
    sa, vax = g0[:, 0:8], g0[:, 8:16]
    vay, vaz = g1[:, 0:8], g1[:, 8:16]
    sb, vbx = v0[:, 0:8], v0[:, 8:16]
    vby, vbz = v1[:, 0:8], v1[:, 8:16]

    s1 = sa * sb
    s2 = (vax * vbx + vay * vby + vaz * vbz) * (1.0 / math.sqrt(3.0))
    s12 = jnp.concatenate([s1, s2], axis=1)

    h = (jnp.dot(x, w2ax_ref[...], preferred_element_type=jnp.float32)
         + jnp.dot(s12, w2as_ref[...], preferred_element_type=jnp.float32))
    h = h * jax.nn.sigmoid(h)
    h = jnp.dot(h, w2b_ref[...], preferred_element_type=jnp.float32)
    h = h * jax.nn.sigmoid(h)
    h = jnp.dot(h, w2c_ref[...], preferred_element_type=jnp.float32)

    r = r_ref[...]
    d2 = jnp.sum(r * r, axis=1, keepdims=True)
    d2 = jnp.where(d2 == 0.0, 1.0, d2)
    d = jnp.sqrt(d2)
    d6 = d2 * d2 * d2
    env = 1.0 - 28.0 * d6 + 48.0 * d6 * d - 21.0 * d6 * d2
    xo_ref[...] = env * h

    wv2 = wv2_ref[...]
    vox_ref[...] = jnp.dot(jnp.concatenate([sa * vbx, vax * sb], axis=1), wv2,
                           preferred_element_type=jnp.float32)
    voy_ref[...] = jnp.dot(jnp.concatenate([sa * vby, vay * sb], axis=1), wv2,
                           preferred_element_type=jnp.float32)
    voz_ref[...] = jnp.dot(jnp.concatenate([sa * vbz, vaz * sb], axis=1), wv2,
                           preferred_element_type=jnp.float32)


def _phase_b(x, g0, g1, v0, v1, r, w2ax, w2as, w2b, w2c, wv2):
    grid = (N_EDGES // BLK,)
    blk = lambda w: pl.BlockSpec((BLK, w), lambda i: (i, 0))
    full = lambda a, b: pl.BlockSpec((a, b), lambda i: (0, 0))
    return pl.pallas_call(
        _phase_b_body,
        grid=grid,
        in_specs=[
            blk(FEAT), blk(16), blk(16), blk(16), blk(16), blk(3),
            full(FEAT, FEAT), full(16, FEAT), full(FEAT, FEAT),
            full(FEAT, FEAT), full(16, 16),
        ],
        out_specs=[blk(FEAT), blk(16), blk(16), blk(16)],
        out_shape=[
            jax.ShapeDtypeStruct((N_EDGES, FEAT), jnp.float32),
            jax.ShapeDtypeStruct((N_EDGES, 16), jnp.float32),
            jax.ShapeDtypeStruct((N_EDGES, 16), jnp.float32),
            jax.ShapeDtypeStruct((N_EDGES, 16), jnp.float32),
        ],
    )(x, g0, g1, v0, v1, r, w2ax, w2as, w2b, w2c, wv2)


def kernel(x, V, r, Y, edge_src, W1, W2a, W2b, W2c, Wv):
    f = jnp.float32
    w1t = jnp.tile(W1, (1, 4)) * (1.0 / (math.sqrt(FEAT)
                                         * math.sqrt(NUM_NEIGHBORS)))
    scale_a = 1.0 / math.sqrt(float(FEAT + 2 * N_MUL))
    w2ax = W2a[:FEAT] * scale_a
    w2as = jnp.concatenate([W2a[FEAT::2], W2a[FEAT + 1::2]], axis=0) * scale_a
    w2b = W2b * (1.0 / math.sqrt(float(W2b.shape[0])))
    w2c = W2c * (1.0 / math.sqrt(float(W2c.shape[0])))
    wv2 = jnp.concatenate([Wv[0::2], Wv[1::2]], axis=0) * (
        1.0 / math.sqrt(float(2 * N_MUL)))

    vt = jnp.transpose(V, (0, 2, 1)).reshape(N_EDGES, 4 * N_MUL)
    v0 = vt[:, 0:16]
    v1 = vt[:, 16:32]

    wy0, wy1 = _phase_a(x.astype(f), Y.astype(f), w1t.astype(f))

    wy = jnp.concatenate([wy0, wy1], axis=1)
    ns = jax.ops.segment_sum(wy, edge_src, num_segments=N_NODES)
    g = ns[edge_src]
    g0, g1 = g[:, 0:16], g[:, 16:32]

    x_out, vox, voy, voz = _phase_b(
        x.astype(f), g0, g1, v0, v1, r.astype(f),
        w2ax, w2as, w2b, w2c, wv2)

    v_out = jnp.stack([vox, voy, voz], axis=-1).reshape(
        N_EDGES, 2 * N_MUL, 3)
    return (x_out, v_out)
